# baseline (speedup 1.0000x reference)
import functools

import jax
import jax.numpy as jnp
from jax import lax
from jax.experimental import pallas as pl
from jax.experimental.pallas import tpu as pltpu
from jax.experimental.pallas import tpu_sc as plsc

N = 10000
DIN = 128
H = 8
CH = 64
HD = H * CH

NPAD = 10240
DUMMY = N
E = 320000
ET = E + N
NW = 32
K = 16
EW = 10320
CHUNKS = EW // K
ETP = NW * EW
NSUB = 16
ZR = NPAD // NSUB
ZB = 8

_f32 = jnp.float32
_mesh = plsc.VectorSubcoreMesh(
    core_axis_name="c", subcore_axis_name="s", num_cores=2, num_subcores=NSUB
)


_GDN = lax.GatherDimensionNumbers(
    offset_dims=(), collapsed_slice_dims=(0,), start_index_map=(0,))


def _take(v, idx):
    return lax.gather(v, idx[:, None], _GDN, slice_sizes=(1,),
                      mode=lax.GatherScatterMode.PROMISE_IN_BOUNDS)


def _lanesum(v, lane):
    for s in (1, 2, 4, 8):
        v = v + _take(v, lane ^ s)
    return v


_BM = 256
_GRID = NPAD // _BM


def _mm1_body(x_ref, wl_ref, bl_ref, wr_ref, br_ref, xl_ref, xr_ref):
    xb = x_ref[...]
    xl_ref[...] = jnp.dot(xb, wl_ref[...], preferred_element_type=_f32) + bl_ref[...]
    xr_ref[...] = jnp.dot(xb, wr_ref[...], preferred_element_type=_f32) + br_ref[...]


def _mm1(xp, wlT, bl, wrT, br):
    return pl.pallas_call(
        _mm1_body,
        grid=(_GRID,),
        in_specs=[
            pl.BlockSpec((_BM, DIN), lambda i: (i, 0)),
            pl.BlockSpec((DIN, HD), lambda i: (0, 0)),
            pl.BlockSpec((1, HD), lambda i: (0, 0)),
            pl.BlockSpec((DIN, HD), lambda i: (0, 0)),
            pl.BlockSpec((1, HD), lambda i: (0, 0)),
        ],
        out_specs=[
            pl.BlockSpec((_BM, HD), lambda i: (i, 0)),
            pl.BlockSpec((_BM, HD), lambda i: (i, 0)),
        ],
        out_shape=[
            jax.ShapeDtypeStruct((NPAD, HD), _f32),
            jax.ShapeDtypeStruct((NPAD, HD), _f32),
        ],
    )(xp, wlT, bl, wrT, br)


def _inv_body(a_ref, b_ref, o_ref):
    o_ref[...] = 1.0 / (a_ref[...] + b_ref[...] + 1e-16)


def _inv(a, b):
    return pl.pallas_call(
        _inv_body,
        grid=(8,),
        in_specs=[
            pl.BlockSpec((NPAD // 8, 128), lambda i: (i, 0)),
            pl.BlockSpec((NPAD // 8, 128), lambda i: (i, 0)),
        ],
        out_specs=pl.BlockSpec((NPAD // 8, 128), lambda i: (i, 0)),
        out_shape=jax.ShapeDtypeStruct((NPAD, 128), _f32),
    )(a, b)


def _mm2_body(p0_ref, p1_ref, b1_ref, x_ref, wla_ref, wlb_ref, bl_ref,
              wra_ref, wrb_ref, br_ref, xl_ref, xr_ref):
    h1 = p0_ref[...] + p1_ref[...] + b1_ref[...]
    h1 = jnp.where(h1 > 0, h1, jnp.exp(h1) - 1.0)
    xb = x_ref[...]
    xl_ref[...] = (
        jnp.dot(xb, wla_ref[...], preferred_element_type=_f32)
        + jnp.dot(h1, wlb_ref[...], preferred_element_type=_f32)
        + bl_ref[...]
    )
    xr_ref[...] = (
        jnp.dot(xb, wra_ref[...], preferred_element_type=_f32)
        + jnp.dot(h1, wrb_ref[...], preferred_element_type=_f32)
        + br_ref[...]
    )


def _mm2(p0, p1, b1, xp, wlaT, wlbT, bl, wraT, wrbT, br):
    return pl.pallas_call(
        _mm2_body,
        grid=(_GRID,),
        in_specs=[
            pl.BlockSpec((_BM, CH), lambda i: (i, 0)),
            pl.BlockSpec((_BM, CH), lambda i: (i, 0)),
            pl.BlockSpec((1, CH), lambda i: (0, 0)),
            pl.BlockSpec((_BM, DIN), lambda i: (i, 0)),
            pl.BlockSpec((DIN, HD), lambda i: (0, 0)),
            pl.BlockSpec((CH, HD), lambda i: (0, 0)),
            pl.BlockSpec((1, HD), lambda i: (0, 0)),
            pl.BlockSpec((DIN, HD), lambda i: (0, 0)),
            pl.BlockSpec((CH, HD), lambda i: (0, 0)),
            pl.BlockSpec((1, HD), lambda i: (0, 0)),
        ],
        out_specs=[
            pl.BlockSpec((_BM, HD), lambda i: (i, 0)),
            pl.BlockSpec((_BM, HD), lambda i: (i, 0)),
        ],
        out_shape=[
            jax.ShapeDtypeStruct((NPAD, HD), _f32),
            jax.ShapeDtypeStruct((NPAD, HD), _f32),
        ],
    )(p0, p1, b1, xp, wlaT, wlbT, bl, wraT, wrbT, br)


def _final_body(p0_ref, p1_ref, b2_ref, o_ref):
    v = p0_ref[...] + p1_ref[...] + b2_ref[...]
    v = v - jnp.max(v, axis=1, keepdims=True)
    lse = jnp.log(jnp.sum(jnp.exp(v), axis=1, keepdims=True))
    o_ref[...] = v - lse


def _final(p0, p1, b2):
    return pl.pallas_call(
        _final_body,
        grid=(_GRID,),
        in_specs=[
            pl.BlockSpec((_BM, CH), lambda i: (i, 0)),
            pl.BlockSpec((_BM, CH), lambda i: (i, 0)),
            pl.BlockSpec((1, CH), lambda i: (0, 0)),
        ],
        out_specs=pl.BlockSpec((_BM, CH), lambda i: (i, 0)),
        out_shape=jax.ShapeDtypeStruct((NPAD, CH), _f32),
    )(p0, p1, b2)


@functools.partial(
    pl.kernel,
    out_type=(
        jax.ShapeDtypeStruct((ETP * 16,), _f32),
        jax.ShapeDtypeStruct((2, NPAD, 128), _f32),
    ),
    mesh=_mesh,
    scratch_types=[
        pltpu.VMEM((K,), jnp.int32),
        pltpu.VMEM((K,), jnp.int32),
        pltpu.VMEM((K, HD), _f32),
        pltpu.VMEM((K, HD), _f32),
        pltpu.VMEM((K * 16,), _f32),
        pltpu.VMEM((K, 128), _f32),
        pltpu.VMEM((HD,), _f32),
        pltpu.VMEM((ZB, 128), _f32),
        pltpu.VMEM_SHARED((NPAD, 128), _f32),
        pltpu.SemaphoreType.DMA,
        pltpu.SemaphoreType.DMA,
    ],
)
def _sweep1(xl_hbm, xr_hbm, src_hbm, dst_hbm, att_hbm, p_hbm, ssum_hbm,
            srcv, dstv, xlr, xrr, pbuf, pwide, attv, zbuf, acc, sem1, sem2):
    cid = lax.axis_index("c")
    sid = lax.axis_index("s")
    wid = sid * 2 + cid
    lane = jnp.arange(16, dtype=jnp.int32)
    zv = jnp.zeros((16,), _f32)

    pltpu.sync_copy(att_hbm, attv)

    def _zero(i, _):
        for j in range(8):
            zbuf[i, pl.ds(j * 16, 16)] = zv
        return 0

    lax.fori_loop(0, ZB, _zero, 0)

    def _pwz(i, _):
        for j in range(8):
            pwide[i, pl.ds(j * 16, 16)] = zv
        return 0

    lax.fori_loop(0, K, _pwz, 0)

    def _zcopy(k, _):
        pltpu.sync_copy(zbuf, acc.at[pl.ds(sid * ZR + k * ZB, ZB)])
        return 0

    lax.fori_loop(0, ZR // ZB, _zcopy, 0)
    plsc.subcore_barrier()

    def _chunk(ci, _):
        base = wid * EW + ci * K
        pltpu.sync_copy(src_hbm.at[pl.ds(base, K)], srcv)
        pltpu.sync_copy(dst_hbm.at[pl.ds(base, K)], dstv)
        pltpu.async_copy(xl_hbm.at[srcv], xlr, sem1).wait()
        pltpu.async_copy(xr_hbm.at[dstv], xrr, sem2).wait()

        def _edge(e, _):
            lv = zv
            for h in range(H):
                accv = zv
                for j in range(CH // 16):
                    off = h * CH + j * 16
                    t = xlr[e, pl.ds(off, 16)] + xrr[e, pl.ds(off, 16)]
                    t = jnp.where(t > 0, t, 0.2 * t)
                    accv = accv + t * attv[pl.ds(off, 16)]
                lv = jnp.where(lane == h, _lanesum(accv, lane), lv)
            pv = jnp.where(lane < H, jnp.exp(lv), 0.0)
            pbuf[pl.ds(e * 16, 16)] = pv
            pwide[e, pl.ds(0, 16)] = pv
            return 0

        lax.fori_loop(0, K, _edge, 0)
        pltpu.sync_copy(pbuf, p_hbm.at[pl.ds(base * 16, K * 16)])
        pltpu.sync_copy(pwide, acc.at[dstv], add=True)
        return 0

    lax.fori_loop(0, CHUNKS, _chunk, 0)
    plsc.subcore_barrier()
    pltpu.sync_copy(acc.at[pl.ds(sid * ZR, ZR)], ssum_hbm.at[cid, pl.ds(sid * ZR, ZR)])


@functools.partial(
    pl.kernel,
    out_type=jax.ShapeDtypeStruct((2, NPAD, CH), _f32),
    mesh=_mesh,
    scratch_types=[
        pltpu.VMEM((K,), jnp.int32),
        pltpu.VMEM((K,), jnp.int32),
        pltpu.VMEM((K, HD), _f32),
        pltpu.VMEM((K, 16), _f32),
        pltpu.VMEM((K, 128), _f32),
        pltpu.VMEM((K, CH), _f32),
        pltpu.VMEM((ZB, CH), _f32),
        pltpu.VMEM_SHARED((NPAD, CH), _f32),
        pltpu.SemaphoreType.DMA,
        pltpu.SemaphoreType.DMA,
        pltpu.SemaphoreType.DMA,
    ],
)
def _sweep2(xl_hbm, src_hbm, dst_hbm, p_hbm, inv_hbm, out_hbm,
            srcv, dstv, xlr, pbuf, invr, cbuf, zbuf, acc, sem1, sem2, sem3):
    cid = lax.axis_index("c")
    sid = lax.axis_index("s")
    wid = sid * 2 + cid

    def _zero(i, _):
        for j in range(CH // 16):
            zbuf[i, pl.ds(j * 16, 16)] = jnp.zeros((16,), _f32)
        return 0

    lax.fori_loop(0, ZB, _zero, 0)

    def _zcopy(k, _):
        pltpu.sync_copy(zbuf, acc.at[pl.ds(sid * ZR + k * ZB, ZB)])
        return 0

    lax.fori_loop(0, ZR // ZB, _zcopy, 0)
    plsc.subcore_barrier()

    def _chunk(ci, _):
        base = wid * EW + ci * K
        pltpu.sync_copy(src_hbm.at[pl.ds(base, K)], srcv)
        pltpu.sync_copy(dst_hbm.at[pl.ds(base, K)], dstv)
        pltpu.async_copy(xl_hbm.at[srcv], xlr, sem1).wait()
        pltpu.sync_copy(p_hbm.at[pl.ds(base, K)], pbuf)
        pltpu.async_copy(inv_hbm.at[dstv], invr, sem2).wait()

        def _edge(e, _):
            av = pbuf[e] * invr[e, pl.ds(0, 16)] * (1.0 / H)
            acc0 = jnp.zeros((16,), _f32)
            acc1 = jnp.zeros((16,), _f32)
            acc2 = jnp.zeros((16,), _f32)
            acc3 = jnp.zeros((16,), _f32)
            for h in range(H):
                a = _take(av, jnp.full((16,), h, jnp.int32))
                off = h * CH
                acc0 = acc0 + a * xlr[e, pl.ds(off, 16)]
                acc1 = acc1 + a * xlr[e, pl.ds(off + 16, 16)]
                acc2 = acc2 + a * xlr[e, pl.ds(off + 32, 16)]
                acc3 = acc3 + a * xlr[e, pl.ds(off + 48, 16)]
            cbuf[e, pl.ds(0, 16)] = acc0
            cbuf[e, pl.ds(16, 16)] = acc1
            cbuf[e, pl.ds(32, 16)] = acc2
            cbuf[e, pl.ds(48, 16)] = acc3
            return 0

        lax.fori_loop(0, K, _edge, 0)
        pltpu.sync_copy(cbuf, acc.at[dstv], add=True)
        return 0

    lax.fori_loop(0, CHUNKS, _chunk, 0)
    plsc.subcore_barrier()
    pltpu.sync_copy(acc.at[pl.ds(sid * ZR, ZR)], out_hbm.at[cid, pl.ds(sid * ZR, ZR)])


def kernel(x, edge_index, Wl1, bl1, Wr1, br1, att1, bias1,
           Wl2, bl2, Wr2, br2, att2, bias2):
    xp = jnp.pad(x, ((0, NPAD - N), (0, 0)))
    loop = jnp.arange(N, dtype=edge_index.dtype)
    srcp = jnp.concatenate(
        [edge_index[0], loop, jnp.full((ETP - ET,), DUMMY, edge_index.dtype)])
    dstp = jnp.concatenate(
        [edge_index[1], loop, jnp.full((ETP - ET,), DUMMY, edge_index.dtype)])

    def _jnp_sweep2(xl, p, inv):
        pr = p.reshape(ETP, 16)
        alpha = pr[:ET, :H] * inv[dstp[:ET], :H] * (1.0 / H)
        msg = xl[srcp[:ET]].reshape(ET, H, CH) * alpha[:, :, None]
        return jax.ops.segment_sum(msg.sum(axis=1), dstp[:ET], num_segments=NPAD)

    xl1, xr1 = _mm1(xp, Wl1.T, bl1.reshape(1, HD), Wr1.T, br1.reshape(1, HD))
    p1, ss1 = _sweep1(xl1, xr1, srcp, dstp, att1.reshape(HD))
    inv1 = _inv(ss1[0], ss1[1])
    o1 = _jnp_sweep2(xl1, p1, inv1)

    xl2, xr2 = _mm2(o1, jnp.zeros_like(o1), bias1.reshape(1, CH), xp,
                    Wl2[:, :DIN].T, Wl2[:, DIN:].T, bl2.reshape(1, HD),
                    Wr2[:, :DIN].T, Wr2[:, DIN:].T, br2.reshape(1, HD))
    p2, ss2 = _sweep1(xl2, xr2, srcp, dstp, att2.reshape(HD))
    inv2 = _inv(ss2[0], ss2[1])
    o2 = _jnp_sweep2(xl2, p2, inv2)

    out = _final(o2, jnp.zeros_like(o2), bias2.reshape(1, CH))
    return out[:N]


# speedup vs baseline: 29.1116x; 29.1116x over previous
import functools

import jax
import jax.numpy as jnp
from jax import lax
from jax.experimental import pallas as pl
from jax.experimental.pallas import tpu as pltpu
from jax.experimental.pallas import tpu_sc as plsc

N = 10000
DIN = 128
H = 8
CH = 64
HD = H * CH

NPAD = 10240
DUMMY = N
E = 320000
ET = E + N
NW = 32
K = 16
EW = 10320
CHUNKS = EW // K
ETP = NW * EW
NSUB = 16
ZR = NPAD // NSUB
ZB = 8

_f32 = jnp.float32
_mesh = plsc.VectorSubcoreMesh(
    core_axis_name="c", subcore_axis_name="s", num_cores=2, num_subcores=NSUB
)


_GDN = lax.GatherDimensionNumbers(
    offset_dims=(), collapsed_slice_dims=(0,), start_index_map=(0,))


def _take(v, idx):
    return lax.gather(v, idx[:, None], _GDN, slice_sizes=(1,),
                      mode=lax.GatherScatterMode.PROMISE_IN_BOUNDS)


def _lanesum(v, lane):
    for s in (1, 2, 4, 8):
        v = v + _take(v, lane ^ s)
    return v


_BM = 256
_GRID = NPAD // _BM


def _mm1_body(x_ref, wl_ref, bl_ref, wr_ref, br_ref, xl_ref, xr_ref):
    xb = x_ref[...]
    xl_ref[...] = jnp.dot(xb, wl_ref[...], preferred_element_type=_f32) + bl_ref[...]
    xr_ref[...] = jnp.dot(xb, wr_ref[...], preferred_element_type=_f32) + br_ref[...]


def _mm1(xp, wlT, bl, wrT, br):
    return pl.pallas_call(
        _mm1_body,
        grid=(_GRID,),
        in_specs=[
            pl.BlockSpec((_BM, DIN), lambda i: (i, 0)),
            pl.BlockSpec((DIN, HD), lambda i: (0, 0)),
            pl.BlockSpec((1, HD), lambda i: (0, 0)),
            pl.BlockSpec((DIN, HD), lambda i: (0, 0)),
            pl.BlockSpec((1, HD), lambda i: (0, 0)),
        ],
        out_specs=[
            pl.BlockSpec((_BM, HD), lambda i: (i, 0)),
            pl.BlockSpec((_BM, HD), lambda i: (i, 0)),
        ],
        out_shape=[
            jax.ShapeDtypeStruct((NPAD, HD), _f32),
            jax.ShapeDtypeStruct((NPAD, HD), _f32),
        ],
    )(xp, wlT, bl, wrT, br)


def _inv_body(a_ref, b_ref, o_ref):
    o_ref[...] = 1.0 / (a_ref[...] + b_ref[...] + 1e-16)


def _inv(a, b):
    return pl.pallas_call(
        _inv_body,
        grid=(8,),
        in_specs=[
            pl.BlockSpec((NPAD // 8, 128), lambda i: (i, 0)),
            pl.BlockSpec((NPAD // 8, 128), lambda i: (i, 0)),
        ],
        out_specs=pl.BlockSpec((NPAD // 8, 128), lambda i: (i, 0)),
        out_shape=jax.ShapeDtypeStruct((NPAD, 128), _f32),
    )(a, b)


def _mm2_body(p0_ref, p1_ref, b1_ref, x_ref, wla_ref, wlb_ref, bl_ref,
              wra_ref, wrb_ref, br_ref, xl_ref, xr_ref):
    h1 = p0_ref[...] + p1_ref[...] + b1_ref[...]
    h1 = jnp.where(h1 > 0, h1, jnp.exp(h1) - 1.0)
    xb = x_ref[...]
    xl_ref[...] = (
        jnp.dot(xb, wla_ref[...], preferred_element_type=_f32)
        + jnp.dot(h1, wlb_ref[...], preferred_element_type=_f32)
        + bl_ref[...]
    )
    xr_ref[...] = (
        jnp.dot(xb, wra_ref[...], preferred_element_type=_f32)
        + jnp.dot(h1, wrb_ref[...], preferred_element_type=_f32)
        + br_ref[...]
    )


def _mm2(p0, p1, b1, xp, wlaT, wlbT, bl, wraT, wrbT, br):
    return pl.pallas_call(
        _mm2_body,
        grid=(_GRID,),
        in_specs=[
            pl.BlockSpec((_BM, CH), lambda i: (i, 0)),
            pl.BlockSpec((_BM, CH), lambda i: (i, 0)),
            pl.BlockSpec((1, CH), lambda i: (0, 0)),
            pl.BlockSpec((_BM, DIN), lambda i: (i, 0)),
            pl.BlockSpec((DIN, HD), lambda i: (0, 0)),
            pl.BlockSpec((CH, HD), lambda i: (0, 0)),
            pl.BlockSpec((1, HD), lambda i: (0, 0)),
            pl.BlockSpec((DIN, HD), lambda i: (0, 0)),
            pl.BlockSpec((CH, HD), lambda i: (0, 0)),
            pl.BlockSpec((1, HD), lambda i: (0, 0)),
        ],
        out_specs=[
            pl.BlockSpec((_BM, HD), lambda i: (i, 0)),
            pl.BlockSpec((_BM, HD), lambda i: (i, 0)),
        ],
        out_shape=[
            jax.ShapeDtypeStruct((NPAD, HD), _f32),
            jax.ShapeDtypeStruct((NPAD, HD), _f32),
        ],
    )(p0, p1, b1, xp, wlaT, wlbT, bl, wraT, wrbT, br)


def _final_body(p0_ref, p1_ref, b2_ref, o_ref):
    v = p0_ref[...] + p1_ref[...] + b2_ref[...]
    v = v - jnp.max(v, axis=1, keepdims=True)
    lse = jnp.log(jnp.sum(jnp.exp(v), axis=1, keepdims=True))
    o_ref[...] = v - lse


def _final(p0, p1, b2):
    return pl.pallas_call(
        _final_body,
        grid=(_GRID,),
        in_specs=[
            pl.BlockSpec((_BM, CH), lambda i: (i, 0)),
            pl.BlockSpec((_BM, CH), lambda i: (i, 0)),
            pl.BlockSpec((1, CH), lambda i: (0, 0)),
        ],
        out_specs=pl.BlockSpec((_BM, CH), lambda i: (i, 0)),
        out_shape=jax.ShapeDtypeStruct((NPAD, CH), _f32),
    )(p0, p1, b2)


@functools.partial(
    pl.kernel,
    out_type=(
        jax.ShapeDtypeStruct((ETP * 16,), _f32),
        jax.ShapeDtypeStruct((2, NPAD, 128), _f32),
    ),
    mesh=_mesh,
    scratch_types=[
        pltpu.VMEM((K,), jnp.int32),
        pltpu.VMEM((K,), jnp.int32),
        pltpu.VMEM((K, HD), _f32),
        pltpu.VMEM((K, HD), _f32),
        pltpu.VMEM((K * 16,), _f32),
        pltpu.VMEM((K, 128), _f32),
        pltpu.VMEM((HD,), _f32),
        pltpu.VMEM((ZB, 128), _f32),
        pltpu.VMEM_SHARED((NPAD, 128), _f32),
        pltpu.SemaphoreType.DMA,
        pltpu.SemaphoreType.DMA,
    ],
)
def _sweep1(xl_hbm, xr_hbm, src_hbm, dst_hbm, att_hbm, p_hbm, ssum_hbm,
            srcv, dstv, xlr, xrr, pbuf, pwide, attv, zbuf, acc, sem1, sem2):
    cid = lax.axis_index("c")
    sid = lax.axis_index("s")
    wid = sid * 2 + cid
    lane = jnp.arange(16, dtype=jnp.int32)
    zv = jnp.zeros((16,), _f32)

    pltpu.sync_copy(att_hbm, attv)

    def _zero(i, _):
        for j in range(8):
            zbuf[i, pl.ds(j * 16, 16)] = zv
        return 0

    lax.fori_loop(0, ZB, _zero, 0)

    def _pwz(i, _):
        for j in range(8):
            pwide[i, pl.ds(j * 16, 16)] = zv
        return 0

    lax.fori_loop(0, K, _pwz, 0)

    def _zcopy(k, _):
        pltpu.sync_copy(zbuf, acc.at[pl.ds(sid * ZR + k * ZB, ZB)])
        return 0

    lax.fori_loop(0, ZR // ZB, _zcopy, 0)
    plsc.subcore_barrier()

    def _chunk(ci, _):
        base = wid * EW + ci * K
        pltpu.sync_copy(src_hbm.at[pl.ds(base, K)], srcv)
        pltpu.sync_copy(dst_hbm.at[pl.ds(base, K)], dstv)
        pltpu.async_copy(xl_hbm.at[srcv], xlr, sem1).wait()
        pltpu.async_copy(xr_hbm.at[dstv], xrr, sem2).wait()

        def _edge(e, _):
            lv = zv
            for h in range(H):
                accv = zv
                for j in range(CH // 16):
                    off = h * CH + j * 16
                    t = xlr[e, pl.ds(off, 16)] + xrr[e, pl.ds(off, 16)]
                    t = jnp.where(t > 0, t, 0.2 * t)
                    accv = accv + t * attv[pl.ds(off, 16)]
                lv = jnp.where(lane == h, _lanesum(accv, lane), lv)
            pv = jnp.where(lane < H, jnp.exp(lv), 0.0)
            pbuf[pl.ds(e * 16, 16)] = pv
            pwide[e, pl.ds(0, 16)] = pv
            return 0

        lax.fori_loop(0, K, _edge, 0)
        pltpu.sync_copy(pbuf, p_hbm.at[pl.ds(base * 16, K * 16)])
        pltpu.sync_copy(pwide, acc.at[dstv], add=True)
        return 0

    lax.fori_loop(0, CHUNKS, _chunk, 0)
    plsc.subcore_barrier()
    pltpu.sync_copy(acc.at[pl.ds(sid * ZR, ZR)], ssum_hbm.at[cid, pl.ds(sid * ZR, ZR)])


@functools.partial(
    pl.kernel,
    out_type=jax.ShapeDtypeStruct((2, NPAD, 128), _f32),
    mesh=_mesh,
    scratch_types=[
        pltpu.VMEM((K,), jnp.int32),
        pltpu.VMEM((K,), jnp.int32),
        pltpu.VMEM((K, HD), _f32),
        pltpu.VMEM((K * 16,), _f32),
        pltpu.VMEM((K, 128), _f32),
        pltpu.VMEM((K, 128), _f32),
        pltpu.VMEM((ZB, 128), _f32),
        pltpu.VMEM_SHARED((NPAD, 128), _f32),
        pltpu.SemaphoreType.DMA,
        pltpu.SemaphoreType.DMA,
    ],
)
def _sweep2(xl_hbm, src_hbm, dst_hbm, p_hbm, inv_hbm, out_hbm,
            srcv, dstv, xlr, pbuf, invr, cbuf, zbuf, acc, sem1, sem2):
    cid = lax.axis_index("c")
    sid = lax.axis_index("s")
    wid = sid * 2 + cid
    zv = jnp.zeros((16,), _f32)

    def _zero(i, _):
        for j in range(8):
            zbuf[i, pl.ds(j * 16, 16)] = zv
        return 0

    lax.fori_loop(0, ZB, _zero, 0)

    def _cbz(i, _):
        for j in range(8):
            cbuf[i, pl.ds(j * 16, 16)] = zv
        return 0

    lax.fori_loop(0, K, _cbz, 0)

    def _zcopy(k, _):
        pltpu.sync_copy(zbuf, acc.at[pl.ds(sid * ZR + k * ZB, ZB)])
        return 0

    lax.fori_loop(0, ZR // ZB, _zcopy, 0)
    plsc.subcore_barrier()

    def _chunk(ci, _):
        base = wid * EW + ci * K
        pltpu.sync_copy(src_hbm.at[pl.ds(base, K)], srcv)
        pltpu.sync_copy(dst_hbm.at[pl.ds(base, K)], dstv)
        cp1 = pltpu.async_copy(xl_hbm.at[srcv], xlr, sem1)
        cp2 = pltpu.async_copy(inv_hbm.at[dstv], invr, sem2)
        pltpu.sync_copy(p_hbm.at[pl.ds(base * 16, K * 16)], pbuf)
        cp1.wait()
        cp2.wait()

        def _edge(e, _):
            av = pbuf[pl.ds(e * 16, 16)] * invr[e, pl.ds(0, 16)] * (1.0 / H)
            acc0 = jnp.zeros((16,), _f32)
            acc1 = jnp.zeros((16,), _f32)
            acc2 = jnp.zeros((16,), _f32)
            acc3 = jnp.zeros((16,), _f32)
            for h in range(H):
                a = _take(av, jnp.full((16,), h, jnp.int32))
                off = h * CH
                acc0 = acc0 + a * xlr[e, pl.ds(off, 16)]
                acc1 = acc1 + a * xlr[e, pl.ds(off + 16, 16)]
                acc2 = acc2 + a * xlr[e, pl.ds(off + 32, 16)]
                acc3 = acc3 + a * xlr[e, pl.ds(off + 48, 16)]
            cbuf[e, pl.ds(0, 16)] = acc0
            cbuf[e, pl.ds(16, 16)] = acc1
            cbuf[e, pl.ds(32, 16)] = acc2
            cbuf[e, pl.ds(48, 16)] = acc3
            return 0

        lax.fori_loop(0, K, _edge, 0)
        pltpu.sync_copy(cbuf, acc.at[dstv], add=True)
        return 0

    lax.fori_loop(0, CHUNKS, _chunk, 0)
    plsc.subcore_barrier()
    pltpu.sync_copy(acc.at[pl.ds(sid * ZR, ZR)], out_hbm.at[cid, pl.ds(sid * ZR, ZR)])


def kernel(x, edge_index, Wl1, bl1, Wr1, br1, att1, bias1,
           Wl2, bl2, Wr2, br2, att2, bias2):
    xp = jnp.pad(x, ((0, NPAD - N), (0, 0)))
    loop = jnp.arange(N, dtype=edge_index.dtype)
    srcp = jnp.concatenate(
        [edge_index[0], loop, jnp.full((ETP - ET,), DUMMY, edge_index.dtype)])
    dstp = jnp.concatenate(
        [edge_index[1], loop, jnp.full((ETP - ET,), DUMMY, edge_index.dtype)])

    xl1, xr1 = _mm1(xp, Wl1.T, bl1.reshape(1, HD), Wr1.T, br1.reshape(1, HD))
    p1, ss1 = _sweep1(xl1, xr1, srcp, dstp, att1.reshape(HD))
    inv1 = _inv(ss1[0], ss1[1])
    o1 = _sweep2(xl1, srcp, dstp, p1, inv1)

    xl2, xr2 = _mm2(o1[0, :, :CH], o1[1, :, :CH], bias1.reshape(1, CH), xp,
                    Wl2[:, :DIN].T, Wl2[:, DIN:].T, bl2.reshape(1, HD),
                    Wr2[:, :DIN].T, Wr2[:, DIN:].T, br2.reshape(1, HD))
    p2, ss2 = _sweep1(xl2, xr2, srcp, dstp, att2.reshape(HD))
    inv2 = _inv(ss2[0], ss2[1])
    o2 = _sweep2(xl2, srcp, dstp, p2, inv2)

    out = _final(o2[0, :, :CH], o2[1, :, :CH], bias2.reshape(1, CH))
    return out[:N]


# speedup vs baseline: 39.4630x; 1.3556x over previous
import functools

import jax
import jax.numpy as jnp
from jax import lax
from jax.experimental import pallas as pl
from jax.experimental.pallas import tpu as pltpu
from jax.experimental.pallas import tpu_sc as plsc

N = 10000
DIN = 128
H = 8
CH = 64
HD = H * CH

NPAD = 10240
DUMMY = N
E = 320000
ET = E + N
NW = 32
K = 32
EW = 10336
CHUNKS = EW // K
ETP = NW * EW
NSUB = 16
ZR = NPAD // NSUB
ZB = 8

_f32 = jnp.float32
_mesh = plsc.VectorSubcoreMesh(
    core_axis_name="c", subcore_axis_name="s", num_cores=2, num_subcores=NSUB
)


_GDN = lax.GatherDimensionNumbers(
    offset_dims=(), collapsed_slice_dims=(0,), start_index_map=(0,))


def _take(v, idx):
    return lax.gather(v, idx[:, None], _GDN, slice_sizes=(1,),
                      mode=lax.GatherScatterMode.PROMISE_IN_BOUNDS)


def _lanesum(v, lane):
    for s in (1, 2, 4, 8):
        v = v + _take(v, lane ^ s)
    return v


_BM = 256
_GRID = NPAD // _BM


def _mm1_body(x_ref, wl_ref, bl_ref, wr_ref, br_ref, xl_ref, xr_ref):
    xb = x_ref[...]
    xl_ref[...] = jnp.dot(xb, wl_ref[...], preferred_element_type=_f32) + bl_ref[...]
    xr_ref[...] = jnp.dot(xb, wr_ref[...], preferred_element_type=_f32) + br_ref[...]


def _mm1(xp, wlT, bl, wrT, br):
    return pl.pallas_call(
        _mm1_body,
        grid=(_GRID,),
        in_specs=[
            pl.BlockSpec((_BM, DIN), lambda i: (i, 0)),
            pl.BlockSpec((DIN, HD), lambda i: (0, 0)),
            pl.BlockSpec((1, HD), lambda i: (0, 0)),
            pl.BlockSpec((DIN, HD), lambda i: (0, 0)),
            pl.BlockSpec((1, HD), lambda i: (0, 0)),
        ],
        out_specs=[
            pl.BlockSpec((_BM, HD), lambda i: (i, 0)),
            pl.BlockSpec((_BM, HD), lambda i: (i, 0)),
        ],
        out_shape=[
            jax.ShapeDtypeStruct((NPAD, HD), _f32),
            jax.ShapeDtypeStruct((NPAD, HD), _f32),
        ],
    )(xp, wlT, bl, wrT, br)


def _inv_body(a_ref, b_ref, o_ref):
    o_ref[...] = 1.0 / (a_ref[...] + b_ref[...] + 1e-16)


def _inv(a, b):
    return pl.pallas_call(
        _inv_body,
        grid=(8,),
        in_specs=[
            pl.BlockSpec((NPAD // 8, 128), lambda i: (i, 0)),
            pl.BlockSpec((NPAD // 8, 128), lambda i: (i, 0)),
        ],
        out_specs=pl.BlockSpec((NPAD // 8, 128), lambda i: (i, 0)),
        out_shape=jax.ShapeDtypeStruct((NPAD, 128), _f32),
    )(a, b)


def _mm2_body(p0_ref, p1_ref, b1_ref, x_ref, wla_ref, wlb_ref, bl_ref,
              wra_ref, wrb_ref, br_ref, xl_ref, xr_ref):
    h1 = p0_ref[...] + p1_ref[...] + b1_ref[...]
    h1 = jnp.where(h1 > 0, h1, jnp.exp(h1) - 1.0)
    xb = x_ref[...]
    xl_ref[...] = (
        jnp.dot(xb, wla_ref[...], preferred_element_type=_f32)
        + jnp.dot(h1, wlb_ref[...], preferred_element_type=_f32)
        + bl_ref[...]
    )
    xr_ref[...] = (
        jnp.dot(xb, wra_ref[...], preferred_element_type=_f32)
        + jnp.dot(h1, wrb_ref[...], preferred_element_type=_f32)
        + br_ref[...]
    )


def _mm2(p0, p1, b1, xp, wlaT, wlbT, bl, wraT, wrbT, br):
    return pl.pallas_call(
        _mm2_body,
        grid=(_GRID,),
        in_specs=[
            pl.BlockSpec((_BM, CH), lambda i: (i, 0)),
            pl.BlockSpec((_BM, CH), lambda i: (i, 0)),
            pl.BlockSpec((1, CH), lambda i: (0, 0)),
            pl.BlockSpec((_BM, DIN), lambda i: (i, 0)),
            pl.BlockSpec((DIN, HD), lambda i: (0, 0)),
            pl.BlockSpec((CH, HD), lambda i: (0, 0)),
            pl.BlockSpec((1, HD), lambda i: (0, 0)),
            pl.BlockSpec((DIN, HD), lambda i: (0, 0)),
            pl.BlockSpec((CH, HD), lambda i: (0, 0)),
            pl.BlockSpec((1, HD), lambda i: (0, 0)),
        ],
        out_specs=[
            pl.BlockSpec((_BM, HD), lambda i: (i, 0)),
            pl.BlockSpec((_BM, HD), lambda i: (i, 0)),
        ],
        out_shape=[
            jax.ShapeDtypeStruct((NPAD, HD), _f32),
            jax.ShapeDtypeStruct((NPAD, HD), _f32),
        ],
    )(p0, p1, b1, xp, wlaT, wlbT, bl, wraT, wrbT, br)


def _final_body(p0_ref, p1_ref, b2_ref, o_ref):
    v = p0_ref[...] + p1_ref[...] + b2_ref[...]
    v = v - jnp.max(v, axis=1, keepdims=True)
    lse = jnp.log(jnp.sum(jnp.exp(v), axis=1, keepdims=True))
    o_ref[...] = v - lse


def _final(p0, p1, b2):
    return pl.pallas_call(
        _final_body,
        grid=(_GRID,),
        in_specs=[
            pl.BlockSpec((_BM, CH), lambda i: (i, 0)),
            pl.BlockSpec((_BM, CH), lambda i: (i, 0)),
            pl.BlockSpec((1, CH), lambda i: (0, 0)),
        ],
        out_specs=pl.BlockSpec((_BM, CH), lambda i: (i, 0)),
        out_shape=jax.ShapeDtypeStruct((NPAD, CH), _f32),
    )(p0, p1, b2)


@functools.partial(
    pl.kernel,
    out_type=(
        jax.ShapeDtypeStruct((ETP * 16,), _f32),
        jax.ShapeDtypeStruct((2, NPAD, 128), _f32),
    ),
    mesh=_mesh,
    scratch_types=[
        pltpu.VMEM((K,), jnp.int32),
        pltpu.VMEM((K,), jnp.int32),
        pltpu.VMEM((K, HD), _f32),
        pltpu.VMEM((K, HD), _f32),
        pltpu.VMEM((K * 16,), _f32),
        pltpu.VMEM((K, 128), _f32),
        pltpu.VMEM((HD,), _f32),
        pltpu.VMEM((ZB, 128), _f32),
        pltpu.VMEM_SHARED((NPAD, 128), _f32),
        pltpu.SemaphoreType.DMA,
        pltpu.SemaphoreType.DMA,
    ],
)
def _sweep1(xl_hbm, xr_hbm, src_hbm, dst_hbm, att_hbm, p_hbm, ssum_hbm,
            srcv, dstv, xlr, xrr, pbuf, pwide, attv, zbuf, acc, sem1, sem2):
    cid = lax.axis_index("c")
    sid = lax.axis_index("s")
    wid = sid * 2 + cid
    lane = jnp.arange(16, dtype=jnp.int32)
    zv = jnp.zeros((16,), _f32)

    pltpu.sync_copy(att_hbm, attv)

    def _zero(i, _):
        for j in range(8):
            zbuf[i, pl.ds(j * 16, 16)] = zv
        return 0

    lax.fori_loop(0, ZB, _zero, 0)

    def _pwz(i, _):
        for j in range(8):
            pwide[i, pl.ds(j * 16, 16)] = zv
        return 0

    lax.fori_loop(0, K, _pwz, 0)

    def _zcopy(k, _):
        pltpu.sync_copy(zbuf, acc.at[pl.ds(sid * ZR + k * ZB, ZB)])
        return 0

    lax.fori_loop(0, ZR // ZB, _zcopy, 0)
    plsc.subcore_barrier()

    def _chunk(ci, _):
        base = wid * EW + ci * K
        pltpu.sync_copy(src_hbm.at[pl.ds(base, K)], srcv)
        pltpu.sync_copy(dst_hbm.at[pl.ds(base, K)], dstv)
        pltpu.async_copy(xl_hbm.at[srcv], xlr, sem1).wait()
        pltpu.async_copy(xr_hbm.at[dstv], xrr, sem2).wait()

        def _edge(e, _):
            lv = zv
            for h in range(H):
                accv = zv
                for j in range(CH // 16):
                    off = h * CH + j * 16
                    t = xlr[e, pl.ds(off, 16)] + xrr[e, pl.ds(off, 16)]
                    t = jnp.where(t > 0, t, 0.2 * t)
                    accv = accv + t * attv[pl.ds(off, 16)]
                lv = jnp.where(lane == h, _lanesum(accv, lane), lv)
            pv = jnp.where(lane < H, jnp.exp(lv), 0.0)
            pbuf[pl.ds(e * 16, 16)] = pv
            pwide[e, pl.ds(0, 16)] = pv
            return 0

        lax.fori_loop(0, K, _edge, 0)
        pltpu.sync_copy(pbuf, p_hbm.at[pl.ds(base * 16, K * 16)])
        pltpu.sync_copy(pwide, acc.at[dstv], add=True)
        return 0

    lax.fori_loop(0, CHUNKS, _chunk, 0)
    plsc.subcore_barrier()
    pltpu.sync_copy(acc.at[pl.ds(sid * ZR, ZR)], ssum_hbm.at[cid, pl.ds(sid * ZR, ZR)])


@functools.partial(
    pl.kernel,
    out_type=jax.ShapeDtypeStruct((2, NPAD, 128), _f32),
    mesh=_mesh,
    scratch_types=[
        pltpu.VMEM((K,), jnp.int32),
        pltpu.VMEM((K,), jnp.int32),
        pltpu.VMEM((K, HD), _f32),
        pltpu.VMEM((K * 16,), _f32),
        pltpu.VMEM((K, 128), _f32),
        pltpu.VMEM((K, 128), _f32),
        pltpu.VMEM((ZB, 128), _f32),
        pltpu.VMEM_SHARED((NPAD, 128), _f32),
        pltpu.SemaphoreType.DMA,
        pltpu.SemaphoreType.DMA,
    ],
)
def _sweep2(xl_hbm, src_hbm, dst_hbm, p_hbm, inv_hbm, out_hbm,
            srcv, dstv, xlr, pbuf, invr, cbuf, zbuf, acc, sem1, sem2):
    cid = lax.axis_index("c")
    sid = lax.axis_index("s")
    wid = sid * 2 + cid
    zv = jnp.zeros((16,), _f32)

    def _zero(i, _):
        for j in range(8):
            zbuf[i, pl.ds(j * 16, 16)] = zv
        return 0

    lax.fori_loop(0, ZB, _zero, 0)

    def _cbz(i, _):
        for j in range(8):
            cbuf[i, pl.ds(j * 16, 16)] = zv
        return 0

    lax.fori_loop(0, K, _cbz, 0)

    def _zcopy(k, _):
        pltpu.sync_copy(zbuf, acc.at[pl.ds(sid * ZR + k * ZB, ZB)])
        return 0

    lax.fori_loop(0, ZR // ZB, _zcopy, 0)
    plsc.subcore_barrier()

    def _chunk(ci, _):
        base = wid * EW + ci * K
        pltpu.sync_copy(src_hbm.at[pl.ds(base, K)], srcv)
        pltpu.sync_copy(dst_hbm.at[pl.ds(base, K)], dstv)
        cp1 = pltpu.async_copy(xl_hbm.at[srcv], xlr, sem1)
        cp2 = pltpu.async_copy(inv_hbm.at[dstv], invr, sem2)
        pltpu.sync_copy(p_hbm.at[pl.ds(base * 16, K * 16)], pbuf)
        cp1.wait()
        cp2.wait()

        def _edge(e, _):
            av = pbuf[pl.ds(e * 16, 16)] * invr[e, pl.ds(0, 16)] * (1.0 / H)
            acc0 = jnp.zeros((16,), _f32)
            acc1 = jnp.zeros((16,), _f32)
            acc2 = jnp.zeros((16,), _f32)
            acc3 = jnp.zeros((16,), _f32)
            for h in range(H):
                a = _take(av, jnp.full((16,), h, jnp.int32))
                off = h * CH
                acc0 = acc0 + a * xlr[e, pl.ds(off, 16)]
                acc1 = acc1 + a * xlr[e, pl.ds(off + 16, 16)]
                acc2 = acc2 + a * xlr[e, pl.ds(off + 32, 16)]
                acc3 = acc3 + a * xlr[e, pl.ds(off + 48, 16)]
            cbuf[e, pl.ds(0, 16)] = acc0
            cbuf[e, pl.ds(16, 16)] = acc1
            cbuf[e, pl.ds(32, 16)] = acc2
            cbuf[e, pl.ds(48, 16)] = acc3
            return 0

        lax.fori_loop(0, K, _edge, 0)
        pltpu.sync_copy(cbuf, acc.at[dstv], add=True)
        return 0

    lax.fori_loop(0, CHUNKS, _chunk, 0)
    plsc.subcore_barrier()
    pltpu.sync_copy(acc.at[pl.ds(sid * ZR, ZR)], out_hbm.at[cid, pl.ds(sid * ZR, ZR)])


def kernel(x, edge_index, Wl1, bl1, Wr1, br1, att1, bias1,
           Wl2, bl2, Wr2, br2, att2, bias2):
    xp = jnp.pad(x, ((0, NPAD - N), (0, 0)))
    loop = jnp.arange(N, dtype=edge_index.dtype)
    srcp = jnp.concatenate(
        [edge_index[0], loop, jnp.full((ETP - ET,), DUMMY, edge_index.dtype)])
    dstp = jnp.concatenate(
        [edge_index[1], loop, jnp.full((ETP - ET,), DUMMY, edge_index.dtype)])

    xl1, xr1 = _mm1(xp, Wl1.T, bl1.reshape(1, HD), Wr1.T, br1.reshape(1, HD))
    p1, ss1 = _sweep1(xl1, xr1, srcp, dstp, att1.reshape(HD))
    inv1 = _inv(ss1[0], ss1[1])
    o1 = _sweep2(xl1, srcp, dstp, p1, inv1)

    xl2, xr2 = _mm2(o1[0, :, :CH], o1[1, :, :CH], bias1.reshape(1, CH), xp,
                    Wl2[:, :DIN].T, Wl2[:, DIN:].T, bl2.reshape(1, HD),
                    Wr2[:, :DIN].T, Wr2[:, DIN:].T, br2.reshape(1, HD))
    p2, ss2 = _sweep1(xl2, xr2, srcp, dstp, att2.reshape(HD))
    inv2 = _inv(ss2[0], ss2[1])
    o2 = _sweep2(xl2, srcp, dstp, p2, inv2)

    out = _final(o2[0, :, :CH], o2[1, :, :CH], bias2.reshape(1, CH))
    return out[:N]
